# baseline (speedup 1.0000x reference)
import functools

import jax
import jax.numpy as jnp
from jax import lax
from jax.experimental import pallas as pl
from jax.experimental.pallas import tpu as pltpu
from jax.experimental.pallas import tpu_sc as plsc

N = 10000
DIN = 128
HID = 128
DOUT = 16
E = 320000
NP = 10240

NC = 2
NS = 16
NW = NC * NS
EPW = E // NW
K = 125
NB = EPW // K
CH = 16
NCH = NB // CH
RPS = NP // NS
ZR = 32


def _sc_degree_body(dst_hbm, out_hbm, acc, ones_v, zbuf, dst_v):
  c = lax.axis_index("c")
  s = lax.axis_index("s")
  wid = s * NC + c

  pltpu.sync_copy(dst_hbm.at[wid], dst_v)

  def fill(i, _):
    for j in range(8):
      ones_v[i, pl.ds(j * 16, 16)] = jnp.ones((16,), jnp.float32)
    return 0
  lax.fori_loop(0, K, fill, 0)

  def zero(i, _):
    for j in range(8):
      zbuf[i, pl.ds(j * 16, 16)] = jnp.zeros((16,), jnp.float32)
    return 0
  lax.fori_loop(0, ZR, zero, 0)
  for z in range(RPS // ZR):
    pltpu.sync_copy(zbuf, acc.at[pl.ds(s * RPS + z * ZR, ZR)])
  plsc.subcore_barrier()

  def step(b, _):
    pltpu.sync_copy(ones_v, acc.at[dst_v.at[b]], add=True)
    return 0
  lax.fori_loop(0, NB, step, 0)
  plsc.subcore_barrier()

  pltpu.sync_copy(acc.at[pl.ds(s * RPS, RPS)],
                  out_hbm.at[pl.ds(c * NP + s * RPS, RPS)])


_sc_degree = functools.partial(
    pl.kernel,
    out_type=jax.ShapeDtypeStruct((2 * NP, HID), jnp.float32),
    mesh=plsc.VectorSubcoreMesh(core_axis_name="c", subcore_axis_name="s"),
    scratch_types=[
        pltpu.VMEM_SHARED((NP, HID), jnp.float32),
        pltpu.VMEM((K, HID), jnp.float32),
        pltpu.VMEM((ZR, HID), jnp.float32),
        pltpu.VMEM((NB, K), jnp.int32),
    ],
)(_sc_degree_body)


def _sc_scatter_body(g_hbm, src_hbm, dst_hbm, out_hbm,
                     acc, rows0, rows1, zbuf, src_v, dst_v, sem0, sem1):
  c = lax.axis_index("c")
  s = lax.axis_index("s")
  wid = s * NC + c

  def zero(i, _):
    for j in range(8):
      zbuf[i, pl.ds(j * 16, 16)] = jnp.zeros((16,), jnp.float32)
    return 0
  lax.fori_loop(0, ZR, zero, 0)
  for z in range(RPS // ZR):
    pltpu.sync_copy(zbuf, acc.at[pl.ds(s * RPS + z * ZR, ZR)])
  plsc.subcore_barrier()

  for ch in range(NCH):
    pltpu.sync_copy(src_hbm.at[wid, ch], src_v)
    pltpu.sync_copy(dst_hbm.at[wid, ch], dst_v)
    pltpu.async_copy(g_hbm.at[src_v.at[0]], rows0, sem0)

    def pair(p, _):
      b0 = 2 * p
      pltpu.async_copy(g_hbm.at[src_v.at[b0 + 1]], rows1, sem1)
      pltpu.make_async_copy(g_hbm.at[src_v.at[b0]], rows0, sem0).wait()
      pltpu.sync_copy(rows0, acc.at[dst_v.at[b0]], add=True)

      @pl.when(p < CH // 2 - 1)
      def _():
        pltpu.async_copy(g_hbm.at[src_v.at[b0 + 2]], rows0, sem0)

      pltpu.make_async_copy(g_hbm.at[src_v.at[b0 + 1]], rows1, sem1).wait()
      pltpu.sync_copy(rows1, acc.at[dst_v.at[b0 + 1]], add=True)
      return 0
    lax.fori_loop(0, CH // 2, pair, 0)
  plsc.subcore_barrier()

  pltpu.sync_copy(acc.at[pl.ds(s * RPS, RPS)],
                  out_hbm.at[pl.ds(c * NP + s * RPS, RPS)])


_sc_scatter = functools.partial(
    pl.kernel,
    out_type=jax.ShapeDtypeStruct((2 * NP, HID), jnp.float32),
    mesh=plsc.VectorSubcoreMesh(core_axis_name="c", subcore_axis_name="s"),
    scratch_types=[
        pltpu.VMEM_SHARED((NP, HID), jnp.float32),
        pltpu.VMEM((K, HID), jnp.float32),
        pltpu.VMEM((K, HID), jnp.float32),
        pltpu.VMEM((ZR, HID), jnp.float32),
        pltpu.VMEM((CH, K), jnp.int32),
        pltpu.VMEM((CH, K), jnp.int32),
        pltpu.SemaphoreType.DMA,
        pltpu.SemaphoreType.DMA,
    ],
)(_sc_scatter_body)


_RB = 512
_GRID = NP // _RB


def _tc_g_body(x_ref, w1_ref, d0_ref, d1_ref, g_ref, dis_ref):
  deg = d0_ref[:, 0:1] + d1_ref[:, 0:1] + 1.0
  dis = lax.rsqrt(deg)
  h = jnp.dot(x_ref[...], w1_ref[...], preferred_element_type=jnp.float32)
  g_ref[...] = h * dis
  dis_ref[...] = jnp.broadcast_to(dis, (dis.shape[0], 8))


def _tc_g(x, w1, degp):
  return pl.pallas_call(
      _tc_g_body,
      grid=(_GRID,),
      in_specs=[
          pl.BlockSpec((_RB, DIN), lambda i: (i, 0)),
          pl.BlockSpec((DIN, HID), lambda i: (0, 0)),
          pl.BlockSpec((_RB, HID), lambda i: (i, 0)),
          pl.BlockSpec((_RB, HID), lambda i: (i + _GRID, 0)),
      ],
      out_specs=[
          pl.BlockSpec((_RB, HID), lambda i: (i, 0)),
          pl.BlockSpec((_RB, 8), lambda i: (i, 0)),
      ],
      out_shape=[
          jax.ShapeDtypeStruct((NP, HID), jnp.float32),
          jax.ShapeDtypeStruct((NP, 8), jnp.float32),
      ],
  )(x, w1, degp, degp)


def _tc_final_body(p0_ref, p1_ref, g_ref, dis_ref,
                   b1_ref, w2_ref, b2_ref, o_ref):
  dis = dis_ref[:, 0:1]
  o = (p0_ref[...] + p1_ref[...] + g_ref[...]) * dis
  a = jnp.maximum(o + b1_ref[...], 0.0)
  z = jnp.dot(a, w2_ref[...], preferred_element_type=jnp.float32)
  z = z + b2_ref[...]
  m = jnp.max(z, axis=1, keepdims=True)
  ez = jnp.exp(z - m)
  sz = jnp.sum(ez, axis=1, keepdims=True)
  o_ref[...] = z - m - jnp.log(sz)


def _tc_final(p, g, dis8, b1, w2, b2):
  return pl.pallas_call(
      _tc_final_body,
      grid=(_GRID,),
      in_specs=[
          pl.BlockSpec((_RB, HID), lambda i: (i, 0)),
          pl.BlockSpec((_RB, HID), lambda i: (i + _GRID, 0)),
          pl.BlockSpec((_RB, HID), lambda i: (i, 0)),
          pl.BlockSpec((_RB, 8), lambda i: (i, 0)),
          pl.BlockSpec((1, HID), lambda i: (0, 0)),
          pl.BlockSpec((HID, DOUT), lambda i: (0, 0)),
          pl.BlockSpec((1, DOUT), lambda i: (0, 0)),
      ],
      out_specs=pl.BlockSpec((_RB, DOUT), lambda i: (i, 0)),
      out_shape=jax.ShapeDtypeStruct((N, DOUT), jnp.float32),
  )(p, p, g, dis8, b1, w2, b2)


def kernel(x, edge_index, W1, b1, W2, b2):
  x = x.astype(jnp.float32)
  src = edge_index[0].astype(jnp.int32).reshape(NW, NCH, CH, K)
  dst = edge_index[1].astype(jnp.int32).reshape(NW, NB, K)
  src4 = src
  dst4 = dst.reshape(NW, NCH, CH, K)

  degp = _sc_degree(dst)
  g, dis8 = _tc_g(x, W1, degp)
  p = _sc_scatter(g, src4, dst4)
  return _tc_final(p, g, dis8, b1.reshape(1, HID), W2, b2.reshape(1, DOUT))


# speedup vs baseline: 1.0381x; 1.0381x over previous
import functools

import jax
import jax.numpy as jnp
from jax import lax
from jax.experimental import pallas as pl
from jax.experimental.pallas import tpu as pltpu
from jax.experimental.pallas import tpu_sc as plsc

N = 10000
DIN = 128
HID = 128
DOUT = 16
E = 320000
NP = 10240

NC = 2
NS = 16
NW = NC * NS
EPW = E // NW
K = 125
NB = EPW // K
CH = 40
NCH = NB // CH
RPS = NP // NS
ZR = 32


def _sc_degree_body(dst_hbm, out_hbm, acc, ones_v, zbuf, dst_v):
  c = lax.axis_index("c")
  s = lax.axis_index("s")
  wid = s * NC + c

  pltpu.sync_copy(dst_hbm.at[wid], dst_v)

  def fill(i, _):
    for j in range(8):
      ones_v[i, pl.ds(j * 16, 16)] = jnp.ones((16,), jnp.float32)
    return 0
  lax.fori_loop(0, K, fill, 0)

  def zero(i, _):
    for j in range(8):
      zbuf[i, pl.ds(j * 16, 16)] = jnp.zeros((16,), jnp.float32)
    return 0
  lax.fori_loop(0, ZR, zero, 0)
  for z in range(RPS // ZR):
    pltpu.sync_copy(zbuf, acc.at[pl.ds(s * RPS + z * ZR, ZR)])
  plsc.subcore_barrier()

  def step(b, _):
    pltpu.sync_copy(ones_v, acc.at[dst_v.at[b]], add=True)
    return 0
  lax.fori_loop(0, NB, step, 0)
  plsc.subcore_barrier()

  pltpu.sync_copy(acc.at[pl.ds(s * RPS, RPS)],
                  out_hbm.at[pl.ds(c * NP + s * RPS, RPS)])


_sc_degree = functools.partial(
    pl.kernel,
    out_type=jax.ShapeDtypeStruct((2 * NP, HID), jnp.float32),
    mesh=plsc.VectorSubcoreMesh(core_axis_name="c", subcore_axis_name="s"),
    scratch_types=[
        pltpu.VMEM_SHARED((NP, HID), jnp.float32),
        pltpu.VMEM((K, HID), jnp.float32),
        pltpu.VMEM((ZR, HID), jnp.float32),
        pltpu.VMEM((NB, K), jnp.int32),
    ],
)(_sc_degree_body)


def _sc_scatter_body(g_hbm, src_hbm, dst_hbm, out_hbm,
                     acc, rows0, rows1, zbuf, src_v, dst_v, sem0, sem1):
  c = lax.axis_index("c")
  s = lax.axis_index("s")
  wid = s * NC + c

  def zero(i, _):
    for j in range(8):
      zbuf[i, pl.ds(j * 16, 16)] = jnp.zeros((16,), jnp.float32)
    return 0
  lax.fori_loop(0, ZR, zero, 0)
  for z in range(RPS // ZR):
    pltpu.sync_copy(zbuf, acc.at[pl.ds(s * RPS + z * ZR, ZR)])
  plsc.subcore_barrier()

  for ch in range(NCH):
    pltpu.sync_copy(src_hbm.at[wid, ch], src_v)
    pltpu.sync_copy(dst_hbm.at[wid, ch], dst_v)
    pltpu.async_copy(g_hbm.at[src_v.at[0]], rows0, sem0)

    def pair(p, _):
      b0 = 2 * p
      pltpu.async_copy(g_hbm.at[src_v.at[b0 + 1]], rows1, sem1)
      pltpu.make_async_copy(g_hbm.at[src_v.at[b0]], rows0, sem0).wait()
      pltpu.sync_copy(rows0, acc.at[dst_v.at[b0]], add=True)

      @pl.when(p < CH // 2 - 1)
      def _():
        pltpu.async_copy(g_hbm.at[src_v.at[b0 + 2]], rows0, sem0)

      pltpu.make_async_copy(g_hbm.at[src_v.at[b0 + 1]], rows1, sem1).wait()
      pltpu.sync_copy(rows1, acc.at[dst_v.at[b0 + 1]], add=True)
      return 0
    lax.fori_loop(0, CH // 2, pair, 0)
  plsc.subcore_barrier()

  pltpu.sync_copy(acc.at[pl.ds(s * RPS, RPS)],
                  out_hbm.at[pl.ds(c * NP + s * RPS, RPS)])


_sc_scatter = functools.partial(
    pl.kernel,
    out_type=jax.ShapeDtypeStruct((2 * NP, HID), jnp.float32),
    mesh=plsc.VectorSubcoreMesh(core_axis_name="c", subcore_axis_name="s"),
    scratch_types=[
        pltpu.VMEM_SHARED((NP, HID), jnp.float32),
        pltpu.VMEM((K, HID), jnp.float32),
        pltpu.VMEM((K, HID), jnp.float32),
        pltpu.VMEM((ZR, HID), jnp.float32),
        pltpu.VMEM((CH, K), jnp.int32),
        pltpu.VMEM((CH, K), jnp.int32),
        pltpu.SemaphoreType.DMA,
        pltpu.SemaphoreType.DMA,
    ],
)(_sc_scatter_body)


_RB = 512
_GRID = NP // _RB


def _tc_h_body(x_ref, w1_ref, h_ref):
  h_ref[...] = jnp.dot(x_ref[...], w1_ref[...],
                       preferred_element_type=jnp.float32)


def _tc_h(x, w1):
  return pl.pallas_call(
      _tc_h_body,
      grid=(_GRID,),
      in_specs=[
          pl.BlockSpec((_RB, DIN), lambda i: (i, 0)),
          pl.BlockSpec((DIN, HID), lambda i: (0, 0)),
      ],
      out_specs=pl.BlockSpec((_RB, HID), lambda i: (i, 0)),
      out_shape=jax.ShapeDtypeStruct((NP, HID), jnp.float32),
  )(x, w1)


def _tc_g_body(h_ref, d0_ref, d1_ref, g_ref, dis_ref):
  deg = d0_ref[:, 0:1] + d1_ref[:, 0:1] + 1.0
  dis = lax.rsqrt(deg)
  g_ref[...] = h_ref[...] * dis
  dis_ref[...] = jnp.broadcast_to(dis, (dis.shape[0], 8))


def _tc_g(h, degp):
  return pl.pallas_call(
      _tc_g_body,
      grid=(_GRID,),
      in_specs=[
          pl.BlockSpec((_RB, HID), lambda i: (i, 0)),
          pl.BlockSpec((_RB, HID), lambda i: (i, 0)),
          pl.BlockSpec((_RB, HID), lambda i: (i + _GRID, 0)),
      ],
      out_specs=[
          pl.BlockSpec((_RB, HID), lambda i: (i, 0)),
          pl.BlockSpec((_RB, 8), lambda i: (i, 0)),
      ],
      out_shape=[
          jax.ShapeDtypeStruct((NP, HID), jnp.float32),
          jax.ShapeDtypeStruct((NP, 8), jnp.float32),
      ],
  )(h, degp, degp)


def _tc_final_body(p0_ref, p1_ref, g_ref, dis_ref,
                   b1_ref, w2_ref, b2_ref, o_ref):
  dis = dis_ref[:, 0:1]
  o = (p0_ref[...] + p1_ref[...] + g_ref[...]) * dis
  a = jnp.maximum(o + b1_ref[...], 0.0)
  z = jnp.dot(a, w2_ref[...], preferred_element_type=jnp.float32)
  z = z + b2_ref[...]
  m = jnp.max(z, axis=1, keepdims=True)
  ez = jnp.exp(z - m)
  sz = jnp.sum(ez, axis=1, keepdims=True)
  o_ref[...] = z - m - jnp.log(sz)


def _tc_final(p, g, dis8, b1, w2, b2):
  return pl.pallas_call(
      _tc_final_body,
      grid=(_GRID,),
      in_specs=[
          pl.BlockSpec((_RB, HID), lambda i: (i, 0)),
          pl.BlockSpec((_RB, HID), lambda i: (i + _GRID, 0)),
          pl.BlockSpec((_RB, HID), lambda i: (i, 0)),
          pl.BlockSpec((_RB, 8), lambda i: (i, 0)),
          pl.BlockSpec((1, HID), lambda i: (0, 0)),
          pl.BlockSpec((HID, DOUT), lambda i: (0, 0)),
          pl.BlockSpec((1, DOUT), lambda i: (0, 0)),
      ],
      out_specs=pl.BlockSpec((_RB, DOUT), lambda i: (i, 0)),
      out_shape=jax.ShapeDtypeStruct((N, DOUT), jnp.float32),
  )(p, p, g, dis8, b1, w2, b2)


def kernel(x, edge_index, W1, b1, W2, b2):
  x = x.astype(jnp.float32)
  src = edge_index[0].astype(jnp.int32).reshape(NW, NCH, CH, K)
  dst = edge_index[1].astype(jnp.int32).reshape(NW, NB, K)
  src4 = src
  dst4 = dst.reshape(NW, NCH, CH, K)

  h = _tc_h(x, W1)
  degp = _sc_degree(dst)
  g, dis8 = _tc_g(h, degp)
  p = _sc_scatter(g, src4, dst4)
  return _tc_final(p, g, dis8, b1.reshape(1, HID), W2, b2.reshape(1, DOUT))
